# Initial kernel scaffold; baseline (speedup 1.0000x reference)
import jax
import jax.numpy as jnp
from jax.experimental import pallas as pl


def kernel(x, token_table, pos_table):
    raise NotImplementedError("write your pallas kernel here")


# speedup vs baseline: 1.4447x; 1.4447x over previous
import functools

import jax
import jax.numpy as jnp
from jax import lax
from jax.experimental import pallas as pl
from jax.experimental.pallas import tpu as pltpu
from jax.experimental.pallas import tpu_sc as plsc

VOCAB = 1000000
MAXLEN = 200
EMBED = 32
BATCH = 4096

_ROWS = BATCH * MAXLEN
_G = 128

_info = plsc.get_sparse_core_info()
_NC, _NS = _info.num_cores, _info.num_subcores
_NW = _NC * _NS
_B_PER_W = _ROWS // _NW
_CHUNK = 3200
_NCHUNK = _B_PER_W // _CHUNK
_NG = _CHUNK // _G
_SEQ_PER_CHUNK = _CHUNK // MAXLEN


@functools.partial(
    pl.kernel,
    out_type=jax.ShapeDtypeStruct((_ROWS, EMBED), jnp.float32),
    mesh=plsc.VectorSubcoreMesh(core_axis_name="c", subcore_axis_name="s"),
    compiler_params=pltpu.CompilerParams(use_tc_tiling_on_sc=False),
    scratch_types=[
        pltpu.VMEM((_CHUNK,), jnp.int32),
        pltpu.VMEM((_CHUNK, EMBED), jnp.float32),
        pltpu.VMEM((MAXLEN, EMBED), jnp.float32),
        pltpu.SemaphoreType.DMA,
    ],
)
def _emb_kernel(x_hbm, tok_hbm, pos_hbm, out_hbm, idx_v, rows_v, pos_v, sem):
    wid = lax.axis_index("s") * _NC + lax.axis_index("c")
    base = wid * _B_PER_W

    pltpu.sync_copy(pos_hbm, pos_v)

    for c in range(_NCHUNK):
        cb = base + c * _CHUNK
        pltpu.sync_copy(x_hbm.at[pl.ds(cb, _CHUNK)], idx_v)
        copies = []
        for j in range(_NG):
            copies.append(
                pltpu.async_copy(
                    tok_hbm.at[idx_v.at[pl.ds(j * _G, _G)]],
                    rows_v.at[pl.ds(j * _G, _G)],
                    sem,
                )
            )
        for cp in copies:
            cp.wait()

        def _add_pos(l, _):
            pv0 = pos_v[l, pl.ds(0, 16)]
            pv1 = pos_v[l, pl.ds(16, 16)]
            for s in range(_SEQ_PER_CHUNK):
                r = s * MAXLEN + l
                plsc.addupdate(rows_v.at[r, pl.ds(0, 16)], pv0)
                plsc.addupdate(rows_v.at[r, pl.ds(16, 16)], pv1)
            return _

        lax.fori_loop(0, MAXLEN, _add_pos, 0)

        pltpu.sync_copy(rows_v, out_hbm.at[pl.ds(cb, _CHUNK)])


def kernel(x, token_table, pos_table):
    xf = x.reshape(_ROWS).astype(jnp.int32)
    out = _emb_kernel(xf, token_table, pos_table)
    return out.reshape(BATCH, MAXLEN, EMBED)
